# baseline (speedup 1.0000x reference)
import functools

import jax
import jax.numpy as jnp
from jax import lax
from jax.experimental import pallas as pl
from jax.experimental.pallas import tpu as pltpu
from jax.experimental.pallas import tpu_sc as plsc

BATCH = 16384
DIM = 32
NC = 2
NS = 16
NW = NC * NS
BPW = BATCH // NW
CHUNK = 128
NCHUNK = BPW // CHUNK


def _mf_body(u_hbm, i_hbm, uw_hbm, iw_hbm, out_hbm,
             uidx_v, iidx_v, ue_v, ie_v, part_v, out_v, sem_u, sem_i):
    wid = lax.axis_index("s") * NC + lax.axis_index("c")

    pltpu.sync_copy(u_hbm.at[wid], uidx_v)
    pltpu.sync_copy(i_hbm.at[wid], iidx_v)

    copies = []
    for j in range(NCHUNK):
        copies.append(pltpu.async_copy(
            uw_hbm.at[uidx_v.at[j]], ue_v.at[pl.ds(j * CHUNK, CHUNK)], sem_u))
        copies.append(pltpu.async_copy(
            iw_hbm.at[iidx_v.at[j]], ie_v.at[pl.ds(j * CHUNK, CHUNK)], sem_i))
    for c in copies:
        c.wait()

    lanes = lax.iota(jnp.int32, 16)

    def group(g, _):
        base = g * 16
        for r in range(16):
            b = base + r
            p = (ue_v[b, pl.ds(0, 16)] * ie_v[b, pl.ds(0, 16)]
                 + ue_v[b, pl.ds(16, 16)] * ie_v[b, pl.ds(16, 16)])
            part_v[r, pl.ds(0, 16)] = p
        acc = plsc.load_gather(part_v, [lanes, jnp.zeros((16,), jnp.int32)])
        for l in range(1, 16):
            acc = acc + plsc.load_gather(
                part_v, [lanes, jnp.full((16,), l, jnp.int32)])
        out_v[pl.ds(base, 16)] = acc
        return _

    lax.fori_loop(0, BPW // 16, group, 0)

    pltpu.sync_copy(out_v, out_hbm.at[wid])


@jax.jit
def _mf_score(u2, i2, user_weight, item_weight):
    mesh = plsc.VectorSubcoreMesh(core_axis_name="c", subcore_axis_name="s")
    return pl.kernel(
        _mf_body,
        out_type=jax.ShapeDtypeStruct((NW, BPW), jnp.float32),
        mesh=mesh,
        compiler_params=pltpu.CompilerParams(
            needs_layout_passes=False, use_tc_tiling_on_sc=False),
        scratch_types=[
            pltpu.VMEM((NCHUNK, CHUNK), jnp.int32),
            pltpu.VMEM((NCHUNK, CHUNK), jnp.int32),
            pltpu.VMEM((BPW, DIM), jnp.float32),
            pltpu.VMEM((BPW, DIM), jnp.float32),
            pltpu.VMEM((16, 17), jnp.float32),
            pltpu.VMEM((BPW,), jnp.float32),
            pltpu.SemaphoreType.DMA,
            pltpu.SemaphoreType.DMA,
        ],
    )(u2, i2, user_weight, item_weight)


def kernel(u, i, user_weight, item_weight):
    u2 = u.reshape(NW, NCHUNK, CHUNK)
    i2 = i.reshape(NW, NCHUNK, CHUNK)
    out = _mf_score(u2, i2, user_weight, item_weight)
    return out.reshape(BATCH)
